# Initial kernel scaffold; baseline (speedup 1.0000x reference)
import jax
import jax.numpy as jnp
from jax.experimental import pallas as pl


def kernel(x, w1, b1, w2, b2, fw1, fb1, fw2, fb2, dropout_key):
    raise NotImplementedError("write your pallas kernel here")


# speedup vs baseline: 8.3495x; 8.3495x over previous
import numpy as np

import jax
import jax.numpy as jnp
from jax.experimental import pallas as pl
from jax.experimental.pallas import tpu as pltpu

_H0, _W0 = 64, 192
_H1, _W1 = 32, 96
_H2, _W2 = 16, 48
_FEAT = _H2 * _W2
_HID = 500
_NCLS = 8
_KP = 5
_BB = 32
_CW = 256
_CW2 = 128


def _band_basis(width):
    D = np.zeros((_KP, width, 256), np.float32)
    j = np.arange(width)
    pos = j // 2 + (j % 2) * 128
    for dj in range(_KP):
        c = j + dj - 2
        v = (c >= 0) & (c < width)
        D[dj, c[v], pos[v]] = 1.0
    return D


_D1 = _band_basis(_W0)
_D2 = _band_basis(_W1)


def _stack_bands(w, b, basis, width, cw):
    chunks = jnp.tensordot(w, jnp.asarray(basis), axes=[[1], [0]])
    full = jnp.zeros((_KP, cw, 256), jnp.float32).at[:, :width, :].set(chunks)
    full = full.at[2, width, :].set(b[0])
    return full.reshape(_KP * cw, 256).astype(jnp.bfloat16)


def _shift_stack(x3, h, cw, step):
    outs = []
    for di in range(_KP):
        s = step * (di - 2)
        if s < 0:
            t = jnp.pad(x3[:, :h + s, :], ((0, 0), (-s, 0), (0, 0)))
        elif s > 0:
            t = jnp.pad(x3[:, s:, :], ((0, 0), (0, s), (0, 0)))
        else:
            t = x3
        outs.append(t)
    return jnp.concatenate(outs, axis=2).reshape(_BB * h, _KP * cw)


def _conv_kernel(x_ref, band1_ref, band2_ref, out_ref):
    xb = x_ref[...].reshape(_BB, _H0, _W0).astype(jnp.bfloat16)
    xp = jnp.pad(xb, ((0, 0), (0, 0), (0, _CW - _W0)), constant_values=1.0)
    x5 = _shift_stack(xp, _H0, _CW, 1)
    a1 = jnp.dot(x5, band1_ref[...],
                 preferred_element_type=jnp.float32)
    a1 = a1.astype(jnp.bfloat16)
    m1 = jnp.maximum(a1, jnp.pad(a1[1:], ((0, 1), (0, 0))))
    p1 = jnp.maximum(m1[:, :128], m1[:, 128:])
    p1 = jnp.maximum(p1, 0.0)
    lane = jax.lax.broadcasted_iota(jnp.int32, p1.shape, 1)
    p1 = jnp.where(lane >= _W1, jnp.bfloat16(1.0), p1)

    x52 = _shift_stack(p1.reshape(_BB, _H0, _CW2), _H0, _CW2, 2)
    a2 = jnp.dot(x52, band2_ref[...],
                 preferred_element_type=jnp.float32)
    a2 = a2.astype(jnp.bfloat16)
    m2 = jnp.maximum(a2, jnp.pad(a2[2:], ((0, 2), (0, 0))))
    p2 = jnp.maximum(m2[:, :128], m2[:, 128:])
    p2 = jnp.maximum(p2, 0.0)
    p23 = p2.reshape(_BB, _H0, _CW2)
    for i in range(_H2):
        out_ref[:, i * _W2:(i + 1) * _W2] = p23[:, 4 * i, :_W2]


def _fc_kernel(f_ref, mask_ref, w1_ref, fb1_ref, w2_ref, fb2_ref, out_ref):
    h = jnp.dot(f_ref[...], w1_ref[...],
                preferred_element_type=jnp.float32) + fb1_ref[...]
    h = jnp.maximum(h, 0.0) * mask_ref[...]
    out_ref[...] = jnp.dot(h.astype(jnp.bfloat16), w2_ref[...],
                           preferred_element_type=jnp.float32) + fb2_ref[...]


def kernel(x, w1, b1, w2, b2, fw1, fb1, fw2, fb2, dropout_key):
    n = x.shape[0]
    npad = ((n + _BB - 1) // _BB) * _BB
    if npad != n:
        x = jnp.pad(x, ((0, npad - n), (0, 0), (0, 0), (0, 0)))

    band1 = _stack_bands(w1, b1, _D1, _W0, _CW)
    band2 = _stack_bands(w2, b2, _D2, _W1, _CW2)

    macs = npad * _H0 * _KP * 256 * (_CW + _CW2)
    conv_cost = pl.CostEstimate(
        flops=int(2 * macs), transcendentals=0,
        bytes_accessed=int(4 * x.size + 2 * npad * _FEAT
                           + 2 * (band1.size + band2.size)))
    flat = pl.pallas_call(
        _conv_kernel,
        out_shape=jax.ShapeDtypeStruct((npad, _FEAT), jnp.bfloat16),
        grid=(npad // _BB,),
        in_specs=[
            pl.BlockSpec((_BB, 1, _H0, _W0), lambda b: (b, 0, 0, 0)),
            pl.BlockSpec(band1.shape, lambda b: (0, 0)),
            pl.BlockSpec(band2.shape, lambda b: (0, 0)),
        ],
        out_specs=pl.BlockSpec((_BB, _FEAT), lambda b: (b, 0)),
        compiler_params=pltpu.CompilerParams(dimension_semantics=("parallel",)),
        cost_estimate=conv_cost,
    )(x, band1, band2)

    keep = jax.random.bernoulli(jax.random.wrap_key_data(dropout_key),
                                0.5, (n, _HID))
    mask = keep.astype(jnp.float32) * 2.0

    nb = 2 if n % 2 == 0 else 1
    bn = n // nb
    fw1b = fw1.astype(jnp.bfloat16)
    fw2b = fw2.astype(jnp.bfloat16)
    fc_cost = pl.CostEstimate(
        flops=int(2 * n * (_FEAT * _HID + _HID * _NCLS)), transcendentals=0,
        bytes_accessed=int(4 * n * (_HID + _NCLS) + 2 * n * _FEAT
                           + 2 * (_FEAT * _HID + _HID * _NCLS)))
    logits = pl.pallas_call(
        _fc_kernel,
        out_shape=jax.ShapeDtypeStruct((n, _NCLS), jnp.float32),
        grid=(nb,),
        in_specs=[
            pl.BlockSpec((bn, _FEAT), lambda b: (b, 0)),
            pl.BlockSpec((bn, _HID), lambda b: (b, 0)),
            pl.BlockSpec(fw1b.shape, lambda b: (0, 0)),
            pl.BlockSpec(fb1.shape, lambda b: (0, 0)),
            pl.BlockSpec(fw2b.shape, lambda b: (0, 0)),
            pl.BlockSpec(fb2.shape, lambda b: (0, 0)),
        ],
        out_specs=pl.BlockSpec((bn, _NCLS), lambda b: (b, 0)),
        compiler_params=pltpu.CompilerParams(dimension_semantics=("parallel",)),
        cost_estimate=fc_cost,
    )(flat[:n], mask, fw1b, fb1, fw2b, fb2)
    return logits


# speedup vs baseline: 10.2115x; 1.2230x over previous
import numpy as np

import jax
import jax.numpy as jnp
from jax.experimental import pallas as pl
from jax.experimental.pallas import tpu as pltpu

_H0, _W0 = 64, 192
_H1, _W1 = 32, 96
_H2, _W2 = 16, 48
_FEAT = _H2 * _W2
_HID = 500
_NCLS = 8
_KP = 5
_BB = 32
_NLANE = 128
_CW = 256
_CW2 = 128


def _band_basis(width):
    D = np.zeros((_KP, width, 256), np.float32)
    j = np.arange(width)
    pos = j // 2 + (j % 2) * 128
    for dj in range(_KP):
        c = j + dj - 2
        v = (c >= 0) & (c < width)
        D[dj, c[v], pos[v]] = 1.0
    return D


_D1 = _band_basis(_W0)
_D2 = _band_basis(_W1)


def _stack_bands(w, b, basis, width, cw):
    chunks = jnp.tensordot(w, jnp.asarray(basis), axes=[[1], [0]])
    full = jnp.zeros((_KP, cw, 256), jnp.float32).at[:, :width, :].set(chunks)
    full = full.at[2, width, :].set(b[0])
    return full.reshape(_KP * cw, 256).astype(jnp.bfloat16)


def _shift_stack(x3, h, cw, step):
    outs = []
    for di in range(_KP):
        s = step * (di - 2)
        if s < 0:
            t = jnp.pad(x3[:, :h + s, :], ((0, 0), (-s, 0), (0, 0)))
        elif s > 0:
            t = jnp.pad(x3[:, s:, :], ((0, 0), (0, s), (0, 0)))
        else:
            t = x3
        outs.append(t)
    return jnp.concatenate(outs, axis=2).reshape(_BB * h, _KP * cw)


def _conv_block(xb, band1, band2, out_ref, row0):
    xp = jnp.pad(xb, ((0, 0), (0, 0), (0, _CW - _W0)), constant_values=1.0)
    x5 = _shift_stack(xp, _H0, _CW, 1)
    a1 = jnp.dot(x5, band1,
                 preferred_element_type=jnp.float32)
    a1 = a1.astype(jnp.bfloat16)
    m1 = jnp.maximum(a1, jnp.pad(a1[1:], ((0, 1), (0, 0))))
    p1 = jnp.maximum(m1[:, :128], m1[:, 128:])
    p1 = jnp.maximum(p1, 0.0)
    lane = jax.lax.broadcasted_iota(jnp.int32, p1.shape, 1)
    p1 = jnp.where(lane >= _W1, jnp.bfloat16(1.0), p1)

    x52 = _shift_stack(p1.reshape(_BB, _H0, _CW2), _H0, _CW2, 2)
    a2 = jnp.dot(x52, band2,
                 preferred_element_type=jnp.float32)
    a2 = a2.astype(jnp.bfloat16)
    m2 = jnp.maximum(a2, jnp.pad(a2[2:], ((0, 2), (0, 0))))
    p2 = jnp.maximum(m2[:, :128], m2[:, 128:])
    p2 = jnp.maximum(p2, 0.0)
    p23 = p2.reshape(_BB, _H0, _CW2)
    for i in range(_H2):
        out_ref[row0:row0 + _BB, i * _W2:(i + 1) * _W2] = p23[:, 4 * i, :_W2]


def _conv_kernel(x_ref, band1_ref, band2_ref, out_ref):
    xt = x_ref[...].astype(jnp.bfloat16).reshape(_H0 * _W0, _NLANE)
    xb = jnp.swapaxes(xt, 0, 1).reshape(_NLANE, _H0, _W0)
    band1 = band1_ref[...]
    band2 = band2_ref[...]
    for sb in range(_NLANE // _BB):
        _conv_block(xb[sb * _BB:(sb + 1) * _BB], band1, band2,
                    out_ref, sb * _BB)


def _fc_kernel(f_ref, mask_ref, w1_ref, fb1_ref, w2_ref, fb2_ref, out_ref):
    h = jnp.dot(f_ref[...], w1_ref[...],
                preferred_element_type=jnp.float32) + fb1_ref[...]
    h = jnp.maximum(h, 0.0) * mask_ref[...]
    out_ref[...] = jnp.dot(h.astype(jnp.bfloat16), w2_ref[...],
                           preferred_element_type=jnp.float32) + fb2_ref[...]


def kernel(x, w1, b1, w2, b2, fw1, fb1, fw2, fb2, dropout_key):
    n = x.shape[0]
    npad = ((n + _NLANE - 1) // _NLANE) * _NLANE
    if npad != n:
        x = jnp.pad(x, ((0, npad - n), (0, 0), (0, 0), (0, 0)))
    xt = jnp.transpose(x.reshape(npad, _H0, _W0), (1, 2, 0))

    band1 = _stack_bands(w1, b1, _D1, _W0, _CW)
    band2 = _stack_bands(w2, b2, _D2, _W1, _CW2)

    macs = npad * _H0 * _KP * 256 * (_CW + _CW2)
    conv_cost = pl.CostEstimate(
        flops=int(2 * macs), transcendentals=0,
        bytes_accessed=int(4 * x.size + 2 * npad * _FEAT
                           + 2 * (band1.size + band2.size)))
    flat = pl.pallas_call(
        _conv_kernel,
        out_shape=jax.ShapeDtypeStruct((npad, _FEAT), jnp.bfloat16),
        grid=(npad // _NLANE,),
        in_specs=[
            pl.BlockSpec((_H0, _W0, _NLANE), lambda b: (0, 0, b)),
            pl.BlockSpec(band1.shape, lambda b: (0, 0)),
            pl.BlockSpec(band2.shape, lambda b: (0, 0)),
        ],
        out_specs=pl.BlockSpec((_NLANE, _FEAT), lambda b: (b, 0)),
        compiler_params=pltpu.CompilerParams(dimension_semantics=("parallel",)),
        cost_estimate=conv_cost,
    )(xt, band1, band2)

    keep = jax.random.bernoulli(jax.random.wrap_key_data(dropout_key),
                                0.5, (n, _HID))
    mask = keep.astype(jnp.float32) * 2.0

    nb = 2 if n % 2 == 0 else 1
    bn = n // nb
    fw1b = fw1.astype(jnp.bfloat16)
    fw2b = fw2.astype(jnp.bfloat16)
    fc_cost = pl.CostEstimate(
        flops=int(2 * n * (_FEAT * _HID + _HID * _NCLS)), transcendentals=0,
        bytes_accessed=int(4 * n * (_HID + _NCLS) + 2 * n * _FEAT
                           + 2 * (_FEAT * _HID + _HID * _NCLS)))
    logits = pl.pallas_call(
        _fc_kernel,
        out_shape=jax.ShapeDtypeStruct((n, _NCLS), jnp.float32),
        grid=(nb,),
        in_specs=[
            pl.BlockSpec((bn, _FEAT), lambda b: (b, 0)),
            pl.BlockSpec((bn, _HID), lambda b: (b, 0)),
            pl.BlockSpec(fw1b.shape, lambda b: (0, 0)),
            pl.BlockSpec(fb1.shape, lambda b: (0, 0)),
            pl.BlockSpec(fw2b.shape, lambda b: (0, 0)),
            pl.BlockSpec(fb2.shape, lambda b: (0, 0)),
        ],
        out_specs=pl.BlockSpec((bn, _NCLS), lambda b: (b, 0)),
        compiler_params=pltpu.CompilerParams(dimension_semantics=("parallel",)),
        cost_estimate=fc_cost,
    )(flat[:n], mask, fw1b, fb1, fw2b, fb2)
    return logits


# speedup vs baseline: 11.7079x; 1.1465x over previous
import numpy as np

import jax
import jax.numpy as jnp
from jax.experimental import pallas as pl
from jax.experimental.pallas import tpu as pltpu

_H0, _W0 = 64, 192
_H1, _W1 = 32, 96
_H2, _W2 = 16, 48
_FEAT = _H2 * _W2
_HID = 500
_NCLS = 8
_KP = 5
_BB = 32
_NLANE = 128
_CW = 256
_CW2 = 128


def _band_basis(width):
    D = np.zeros((_KP, width, 256), np.float32)
    j = np.arange(width)
    pos = j // 2 + (j % 2) * 128
    for dj in range(_KP):
        c = j + dj - 2
        v = (c >= 0) & (c < width)
        D[dj, c[v], pos[v]] = 1.0
    return D


def _band_basis_p(width, half):
    D = np.zeros((_KP, width, 2 * half), np.float32)
    j = np.arange(width)
    pos = j // 2 + (j % 2) * half
    for dj in range(_KP):
        c = j + dj - 2
        v = (c >= 0) & (c < width)
        D[dj, c[v], pos[v]] = 1.0
    return D


_D1 = _band_basis(_W0)
_D2P = _band_basis_p(_W1, 64)


def _stack_bands(w, b, basis, width, cw):
    chunks = jnp.tensordot(w, jnp.asarray(basis), axes=[[1], [0]])
    full = jnp.zeros((_KP, cw, 256), jnp.float32).at[:, :width, :].set(chunks)
    full = full.at[2, width, :].set(b[0])
    return full.reshape(_KP * cw, 256).astype(jnp.bfloat16)


def _packed_bands2(w, b):
    cp = jnp.tensordot(w, jnp.asarray(_D2P), axes=[[1], [0]])
    full = jnp.zeros((3, 2, 128, 2, 128), jnp.float32)
    for c in range(3):
        for h in range(2):
            for e in range(2):
                di = 2 * c + h - e
                if 0 <= di < _KP:
                    full = full.at[c, h, :_W1, e, :].set(cp[di])
    full = full.reshape(3 * 256, 256)
    full = full.at[256 + _W1, :].set(b[0])
    return full.astype(jnp.bfloat16)


def _shift_stack(x3, h, cw, step, taps=_KP, center=2):
    outs = []
    for di in range(taps):
        s = step * (di - center)
        if s < 0:
            t = jnp.pad(x3[:, :h + s, :], ((0, 0), (-s, 0), (0, 0)))
        elif s > 0:
            t = jnp.pad(x3[:, s:, :], ((0, 0), (0, s), (0, 0)))
        else:
            t = x3
        outs.append(t)
    return jnp.concatenate(outs, axis=2).reshape(_BB * h, taps * cw)


def _conv_block(xb, band1, band2, out_ref, scr_ref, row0):
    xp = jnp.pad(xb, ((0, 0), (0, 0), (0, _CW - _W0)), constant_values=1.0)
    x5 = _shift_stack(xp, _H0, _CW, 1)
    a1 = jnp.dot(x5, band1,
                 preferred_element_type=jnp.float32)
    m1 = jnp.maximum(a1, jnp.pad(a1[1:], ((0, 1), (0, 0))))
    p1 = jnp.maximum(m1[:, :128], m1[:, 128:])
    p1 = jnp.maximum(p1, 0.0)
    lane = jax.lax.broadcasted_iota(jnp.int32, p1.shape, 1)
    p1 = jnp.where(lane >= _W1, 1.0, p1)

    scr_ref[...] = p1
    tp = jnp.concatenate(
        [scr_ref[pl.Slice(0, _BB * _H2, 4), :],
         scr_ref[pl.Slice(2, _BB * _H2, 4), :]], axis=1)
    tp = tp.astype(jnp.bfloat16)
    x52 = _shift_stack(tp.reshape(_BB, _H2, 256), _H2, 256, 1,
                       taps=3, center=1)
    a2 = jnp.dot(x52, band2,
                 preferred_element_type=jnp.float32)
    a2 = a2.astype(jnp.bfloat16)
    f = jnp.maximum(jnp.maximum(a2[:, 0:_W2], a2[:, 64:64 + _W2]),
                    jnp.maximum(a2[:, 128:128 + _W2], a2[:, 192:192 + _W2]))
    f = jnp.maximum(f, 0.0)
    f3 = f.reshape(_BB, _H2, _W2)
    for i in range(_H2):
        out_ref[row0:row0 + _BB, i * _W2:(i + 1) * _W2] = f3[:, i, :]


def _conv_kernel(x_ref, band1_ref, band2_ref, out_ref, scr_ref):
    xt = x_ref[...].astype(jnp.bfloat16).reshape(_H0 * _W0, _NLANE)
    xb = jnp.swapaxes(xt, 0, 1).reshape(_NLANE, _H0, _W0)
    band1 = band1_ref[...]
    band2 = band2_ref[...]
    for sb in range(_NLANE // _BB):
        _conv_block(xb[sb * _BB:(sb + 1) * _BB], band1, band2,
                    out_ref, scr_ref, sb * _BB)


def _fc_kernel(f_ref, mask_ref, w1_ref, fb1_ref, w2_ref, fb2_ref, out_ref):
    h = jnp.dot(f_ref[...], w1_ref[...],
                preferred_element_type=jnp.float32) + fb1_ref[...]
    h = jnp.maximum(h, 0.0) * mask_ref[...]
    out_ref[...] = jnp.dot(h.astype(jnp.bfloat16), w2_ref[...],
                           preferred_element_type=jnp.float32) + fb2_ref[...]


def kernel(x, w1, b1, w2, b2, fw1, fb1, fw2, fb2, dropout_key):
    n = x.shape[0]
    npad = ((n + _NLANE - 1) // _NLANE) * _NLANE
    if npad != n:
        x = jnp.pad(x, ((0, npad - n), (0, 0), (0, 0), (0, 0)))
    xt = jnp.transpose(x.reshape(npad, _H0, _W0), (1, 2, 0))

    band1 = _stack_bands(w1, b1, _D1, _W0, _CW)
    band2 = _packed_bands2(w2, b2)

    macs = npad * _H0 * _KP * 256 * (_CW + _CW2)
    conv_cost = pl.CostEstimate(
        flops=int(2 * macs), transcendentals=0,
        bytes_accessed=int(4 * x.size + 2 * npad * _FEAT
                           + 2 * (band1.size + band2.size)))
    flat = pl.pallas_call(
        _conv_kernel,
        out_shape=jax.ShapeDtypeStruct((npad, _FEAT), jnp.bfloat16),
        grid=(npad // _NLANE,),
        in_specs=[
            pl.BlockSpec((_H0, _W0, _NLANE), lambda b: (0, 0, b)),
            pl.BlockSpec(band1.shape, lambda b: (0, 0)),
            pl.BlockSpec(band2.shape, lambda b: (0, 0)),
        ],
        out_specs=pl.BlockSpec((_NLANE, _FEAT), lambda b: (b, 0)),
        scratch_shapes=[pltpu.VMEM((_BB * _H0, 128), jnp.float32)],
        compiler_params=pltpu.CompilerParams(dimension_semantics=("parallel",)),
        cost_estimate=conv_cost,
    )(xt, band1, band2)

    keep = jax.random.bernoulli(jax.random.wrap_key_data(dropout_key),
                                0.5, (n, _HID))
    mask = keep.astype(jnp.float32) * 2.0

    nb = 2 if n % 2 == 0 else 1
    bn = n // nb
    fw1b = fw1.astype(jnp.bfloat16)
    fw2b = fw2.astype(jnp.bfloat16)
    fc_cost = pl.CostEstimate(
        flops=int(2 * n * (_FEAT * _HID + _HID * _NCLS)), transcendentals=0,
        bytes_accessed=int(4 * n * (_HID + _NCLS) + 2 * n * _FEAT
                           + 2 * (_FEAT * _HID + _HID * _NCLS)))
    logits = pl.pallas_call(
        _fc_kernel,
        out_shape=jax.ShapeDtypeStruct((n, _NCLS), jnp.float32),
        grid=(nb,),
        in_specs=[
            pl.BlockSpec((bn, _FEAT), lambda b: (b, 0)),
            pl.BlockSpec((bn, _HID), lambda b: (b, 0)),
            pl.BlockSpec(fw1b.shape, lambda b: (0, 0)),
            pl.BlockSpec(fb1.shape, lambda b: (0, 0)),
            pl.BlockSpec(fw2b.shape, lambda b: (0, 0)),
            pl.BlockSpec(fb2.shape, lambda b: (0, 0)),
        ],
        out_specs=pl.BlockSpec((bn, _NCLS), lambda b: (b, 0)),
        compiler_params=pltpu.CompilerParams(dimension_semantics=("parallel",)),
        cost_estimate=fc_cost,
    )(flat[:n], mask, fw1b, fb1, fw2b, fb2)
    return logits
